# baseline (speedup 1.0000x reference)
import functools

import jax
import jax.numpy as jnp
from jax import lax
from jax.experimental import pallas as pl
from jax.experimental.pallas import tpu as pltpu
from jax.experimental.pallas import tpu_sc as plsc

_F_BLK = 64


def _sc_lookup(emb_table, F):
    C = emb_table.shape[1]
    mesh = plsc.VectorSubcoreMesh(core_axis_name="c", subcore_axis_name="s")

    @functools.partial(
        pl.kernel,
        mesh=mesh,
        out_type=jax.ShapeDtypeStruct((F, C), jnp.float32),
        scratch_types=[
            pltpu.VMEM((F,), jnp.int32),
            pltpu.VMEM((F, C), jnp.float32),
            pltpu.SemaphoreType.DMA,
        ],
    )
    def k(emb_hbm, out_hbm, idx_v, rows_v, sem):
        wid = lax.axis_index("s") * 2 + lax.axis_index("c")

        @pl.when(wid == 0)
        def _():
            for ch in range(F // 16):
                idx_v[pl.ds(ch * 16, 16)] = (
                    lax.iota(jnp.int32, 16) + ch * 16
                )
            pltpu.async_copy(emb_hbm.at[idx_v], rows_v, sem).wait()
            pltpu.sync_copy(rows_v, out_hbm)

    return k(emb_table)


def _tc_add_kernel(x_ref, fe_ref, o_ref):
    fe = fe_ref[...].T
    o_ref[...] = x_ref[...] + fe[None, :, :, None]


def kernel(x, emb_table):
    b, c, f, t = x.shape
    femap = _sc_lookup(emb_table, f)
    grid = (b, f // _F_BLK)
    return pl.pallas_call(
        _tc_add_kernel,
        grid=grid,
        in_specs=[
            pl.BlockSpec((1, c, _F_BLK, t), lambda i, j: (i, 0, j, 0)),
            pl.BlockSpec((_F_BLK, c), lambda i, j: (j, 0)),
        ],
        out_specs=pl.BlockSpec((1, c, _F_BLK, t), lambda i, j: (i, 0, j, 0)),
        out_shape=jax.ShapeDtypeStruct(x.shape, x.dtype),
    )(x, femap)
